# Initial kernel scaffold; baseline (speedup 1.0000x reference)
import jax
import jax.numpy as jnp
from jax.experimental import pallas as pl


def kernel(feature_u, feature_v, edge_row, edge_col, edge_val, side_feature_u, side_feature_v, W, Q, W_side_u, b_side_u, g_side_u, be_side_u, W_side_v, b_side_v, g_side_v, be_side_v, W_cat_u, b_cat_u, g_cat_u, be_cat_u, W_cat_v, b_cat_v, g_cat_v, be_cat_v):
    raise NotImplementedError("write your pallas kernel here")


# speedup vs baseline: 13.3959x; 13.3959x over previous
import functools

import jax
import jax.numpy as jnp
from jax import lax
from jax.experimental import pallas as pl
from jax.experimental.pallas import tpu as pltpu
from jax.experimental.pallas import tpu_sc as plsc

U = 4096
V = 4096
FD = 8192
HID = 32
R = 5
E = 400000
SIDE = 128
SH = 64
OUT = 64
RH = R * HID
CIN = R * HID * 2 + SH

RI = 512
KB = 2048
NI = U // RI
NK = FD // KB


def _featmm_body(fu_ref, fv_ref, wf_ref, xv_ref, xu_ref, accv, accu):
    k = pl.program_id(1)

    @pl.when(k == 0)
    def _():
        accv[...] = jnp.zeros_like(accv)
        accu[...] = jnp.zeros_like(accu)

    accv[...] += jnp.dot(fu_ref[...], wf_ref[...],
                         preferred_element_type=jnp.float32)
    accu[...] += jnp.dot(fv_ref[...], wf_ref[...],
                         preferred_element_type=jnp.float32)

    @pl.when(k == NK - 1)
    def _():
        xv_ref[...] = accv[...].reshape(RI, R, HID)
        xu_ref[...] = accu[...].reshape(RI, R, HID)


def _featmm(fu, fv, wf):
    return pl.pallas_call(
        _featmm_body,
        grid=(NI, NK),
        in_specs=[
            pl.BlockSpec((RI, KB), lambda i, k: (i, k)),
            pl.BlockSpec((RI, KB), lambda i, k: (i, k)),
            pl.BlockSpec((KB, RH), lambda i, k: (k, 0)),
        ],
        out_specs=[
            pl.BlockSpec((RI, R, HID), lambda i, k: (i, 0, 0)),
            pl.BlockSpec((RI, R, HID), lambda i, k: (i, 0, 0)),
        ],
        out_shape=[
            jax.ShapeDtypeStruct((U, R, HID), jnp.float32),
            jax.ShapeDtypeStruct((V, R, HID), jnp.float32),
        ],
        scratch_shapes=[
            pltpu.VMEM((RI, RH), jnp.float32),
            pltpu.VMEM((RI, RH), jnp.float32),
        ],
    )(fu, fv, wf)


NWORK = 32
NCHUNK = 7
C = 1792
NS = C // 128
EPAD = NWORK * NCHUNK * C
ACC_ROWS = R * U


def _sc_agg_body(xu_ref, xv_ref, er_ref, ec_ref, ev_ref, h_out,
                 rawg, raws, vals, gidx, sidx, rows, zbuf, acc, gsem):
    cid = lax.axis_index("c")
    sid = lax.axis_index("s")
    wid = cid * 16 + sid

    def zloop(i, carry):
        zbuf[i, pl.ds(0, 16)] = jnp.zeros((16,), jnp.float32)
        zbuf[i, pl.ds(16, 16)] = jnp.zeros((16,), jnp.float32)
        return carry

    lax.fori_loop(0, 640, zloop, 0)

    for d in range(2):
        for kk in range(2):
            pltpu.sync_copy(zbuf, acc.at[pl.ds(sid * 1280 + kk * 640, 640)])
        plsc.subcore_barrier()
        for r in range(R):
            gsrc = ec_ref if d == 0 else er_ref
            ssrc = er_ref if d == 0 else ec_ref
            tab = xu_ref if d == 0 else xv_ref

            def chunk_body(ch, carry, gsrc=gsrc, ssrc=ssrc, tab=tab, r=r):
                base = r * EPAD + (wid * NCHUNK + ch) * C
                pltpu.sync_copy(gsrc.at[pl.ds(base, C)], rawg)
                pltpu.sync_copy(ssrc.at[pl.ds(base, C)], raws)
                pltpu.sync_copy(ev_ref.at[pl.ds(base, C)], vals)

                def idx_body(i, icarry):
                    jj = i // 8
                    loff = (i % 8) * 16
                    gidx[jj, pl.ds(loff, 16)] = rawg[pl.ds(i * 16, 16)] * R + r
                    sidx[jj, pl.ds(loff, 16)] = raws[pl.ds(i * 16, 16)] + r * U
                    return icarry

                lax.fori_loop(0, C // 16, idx_body, 0)

                cps = []
                for j in range(NS):
                    cps.append(pltpu.async_copy(
                        tab.at[gidx.at[j]],
                        rows.at[pl.ds(j * 128, 128)], gsem))
                for cp in cps:
                    cp.wait()

                def scale_body(g, scarry):
                    vv = vals[pl.ds(g * 16, 16)]
                    for j in range(16):
                        e = g * 16 + j
                        v = vv[j]
                        rows[e, pl.ds(0, 16)] = rows[e, pl.ds(0, 16)] * v
                        rows[e, pl.ds(16, 16)] = rows[e, pl.ds(16, 16)] * v
                    return scarry

                lax.fori_loop(0, C // 16, scale_body, 0)

                for j in range(NS):
                    pltpu.sync_copy(rows.at[pl.ds(j * 128, 128)],
                                    acc.at[sidx.at[j]], add=True)
                return carry

            lax.fori_loop(0, NCHUNK, chunk_body, 0)

        plsc.subcore_barrier()
        pltpu.sync_copy(acc.at[pl.ds(sid * 1280, 1280)],
                        h_out.at[cid, d, pl.ds(sid * 1280, 1280)])


def _sc_aggregate(xu_flat, xv_flat, er_p, ec_p, ev_p):
    mesh = plsc.VectorSubcoreMesh(core_axis_name="c", subcore_axis_name="s")
    f = functools.partial(
        pl.kernel,
        out_type=jax.ShapeDtypeStruct((2, 2, ACC_ROWS, HID), jnp.float32),
        mesh=mesh,
        scratch_types=[
            pltpu.VMEM((C,), jnp.int32),
            pltpu.VMEM((C,), jnp.int32),
            pltpu.VMEM((C,), jnp.float32),
            pltpu.VMEM((NS, 128), jnp.int32),
            pltpu.VMEM((NS, 128), jnp.int32),
            pltpu.VMEM((C, HID), jnp.float32),
            pltpu.VMEM((640, HID), jnp.float32),
            pltpu.VMEM_SHARED((ACC_ROWS, HID), jnp.float32),
            pltpu.SemaphoreType.DMA,
        ],
        compiler_params=pltpu.CompilerParams(use_tc_tiling_on_sc=False),
    )(_sc_agg_body)
    return f(xu_flat, xv_flat, er_p, ec_p, ev_p)


RT = 1024
NT = U // RT


def _bn_relu(x, g, b):
    mu = jnp.mean(x, axis=0, keepdims=True)
    var = jnp.mean((x - mu) ** 2, axis=0, keepdims=True)
    return jax.nn.relu(g * (x - mu) / jnp.sqrt(var + 1e-5) + b)


def _combine_a_body(h_ref, xv_ref, xu_ref, sfu_ref, sfv_ref,
                    wsu_ref, wsv_ref, wcu_ref, wcv_ref,
                    p0u_ref, p0v_ref, su_ref, sv_ref):
    wcu = wcu_ref[...]
    wcv = wcv_ref[...]
    pre_u = jnp.zeros((RT, OUT), jnp.float32)
    pre_v = jnp.zeros((RT, OUT), jnp.float32)
    for r in range(R):
        hu_r = jax.nn.relu(h_ref[0, 0, r] + h_ref[1, 0, r])
        hv_r = jax.nn.relu(h_ref[0, 1, r] + h_ref[1, 1, r])
        pre_u = pre_u + jnp.dot(hu_r, wcu[r * HID:(r + 1) * HID],
                                preferred_element_type=jnp.float32)
        pre_v = pre_v + jnp.dot(hv_r, wcv[r * HID:(r + 1) * HID],
                                preferred_element_type=jnp.float32)
        pre_u = pre_u + jnp.dot(xv_ref[:, r, :],
                                wcu[RH + r * HID:RH + (r + 1) * HID],
                                preferred_element_type=jnp.float32)
        pre_v = pre_v + jnp.dot(xu_ref[:, r, :],
                                wcv[RH + r * HID:RH + (r + 1) * HID],
                                preferred_element_type=jnp.float32)
    p0u_ref[...] = pre_u
    p0v_ref[...] = pre_v
    su_ref[...] = jnp.dot(sfu_ref[...], wsu_ref[...],
                          preferred_element_type=jnp.float32)
    sv_ref[...] = jnp.dot(sfv_ref[...], wsv_ref[...],
                          preferred_element_type=jnp.float32)


def _combine_a(h5, xv_all, xu_all, sfu, sfv, wsu, wsv, wcu, wcv):
    return pl.pallas_call(
        _combine_a_body,
        grid=(NT,),
        in_specs=[
            pl.BlockSpec((2, 2, R, RT, HID), lambda i: (0, 0, 0, i, 0)),
            pl.BlockSpec((RT, R, HID), lambda i: (i, 0, 0)),
            pl.BlockSpec((RT, R, HID), lambda i: (i, 0, 0)),
            pl.BlockSpec((RT, SIDE), lambda i: (i, 0)),
            pl.BlockSpec((RT, SIDE), lambda i: (i, 0)),
            pl.BlockSpec((SIDE, SH), lambda i: (0, 0)),
            pl.BlockSpec((SIDE, SH), lambda i: (0, 0)),
            pl.BlockSpec((CIN, OUT), lambda i: (0, 0)),
            pl.BlockSpec((CIN, OUT), lambda i: (0, 0)),
        ],
        out_specs=[
            pl.BlockSpec((RT, OUT), lambda i: (i, 0)),
            pl.BlockSpec((RT, OUT), lambda i: (i, 0)),
            pl.BlockSpec((RT, SH), lambda i: (i, 0)),
            pl.BlockSpec((RT, SH), lambda i: (i, 0)),
        ],
        out_shape=[
            jax.ShapeDtypeStruct((U, OUT), jnp.float32),
            jax.ShapeDtypeStruct((V, OUT), jnp.float32),
            jax.ShapeDtypeStruct((U, SH), jnp.float32),
            jax.ShapeDtypeStruct((V, SH), jnp.float32),
        ],
    )(h5, xv_all, xu_all, sfu, sfv, wsu, wsv, wcu, wcv)


def _combine_b_body(p0u_ref, p0v_ref, su_ref, sv_ref,
                    gsu_ref, besu_ref, gsv_ref, besv_ref,
                    wcu_s_ref, wcv_s_ref,
                    gcu_ref, becu_ref, gcv_ref, becv_ref,
                    eu_ref, ev_ref):
    shu = _bn_relu(su_ref[...], gsu_ref[...], besu_ref[...])
    shv = _bn_relu(sv_ref[...], gsv_ref[...], besv_ref[...])
    pre_u = p0u_ref[...] + jnp.dot(shu, wcu_s_ref[...],
                                   preferred_element_type=jnp.float32)
    pre_v = p0v_ref[...] + jnp.dot(shv, wcv_s_ref[...],
                                   preferred_element_type=jnp.float32)
    eu_ref[...] = _bn_relu(pre_u, gcu_ref[...], becu_ref[...])
    ev_ref[...] = _bn_relu(pre_v, gcv_ref[...], becv_ref[...])


def _combine_b(p0u, p0v, su, sv, gsu, besu, gsv, besv,
               wcu_s, wcv_s, gcu, becu, gcv, becv):
    return pl.pallas_call(
        _combine_b_body,
        out_shape=[
            jax.ShapeDtypeStruct((U, OUT), jnp.float32),
            jax.ShapeDtypeStruct((V, OUT), jnp.float32),
        ],
    )(p0u, p0v, su, sv, gsu, besu, gsv, besv,
      wcu_s, wcv_s, gcu, becu, gcv, becv)


TU = 1024
TV = 1024
NU = U // TU
NV = V // TV


def _score_body(eu_ref, q_ref, ev_ref, out_ref, aq):
    j = pl.program_id(2)

    @pl.when(j == 0)
    def _():
        aq[...] = jnp.dot(eu_ref[...], q_ref[0],
                          preferred_element_type=jnp.float32)

    out_ref[0] = lax.dot_general(aq[...], ev_ref[...],
                                 (((1,), (1,)), ((), ())),
                                 preferred_element_type=jnp.float32)


def _score(eu, q, ev):
    return pl.pallas_call(
        _score_body,
        grid=(R, NU, NV),
        in_specs=[
            pl.BlockSpec((TU, OUT), lambda r, i, j: (i, 0)),
            pl.BlockSpec((1, OUT, OUT), lambda r, i, j: (r, 0, 0)),
            pl.BlockSpec((TV, OUT), lambda r, i, j: (j, 0)),
        ],
        out_specs=pl.BlockSpec((1, TU, TV), lambda r, i, j: (r, i, j)),
        out_shape=jax.ShapeDtypeStruct((R, U, V), jnp.float32),
        scratch_shapes=[pltpu.VMEM((TU, OUT), jnp.float32)],
    )(eu, q, ev)


def kernel(feature_u, feature_v, edge_row, edge_col, edge_val,
           side_feature_u, side_feature_v, W, Q,
           W_side_u, b_side_u, g_side_u, be_side_u,
           W_side_v, b_side_v, g_side_v, be_side_v,
           W_cat_u, b_cat_u, g_cat_u, be_cat_u,
           W_cat_v, b_cat_v, g_cat_v, be_cat_v):
    wf = W.transpose(1, 0, 2).reshape(FD, RH)
    xv_all, xu_all = _featmm(feature_u, feature_v, wf)

    padn = EPAD - E
    pidx = jnp.arange(padn, dtype=jnp.int32) % U
    er_p = jnp.concatenate(
        [edge_row, jnp.broadcast_to(pidx, (R, padn))], axis=1)
    ec_p = jnp.concatenate(
        [edge_col, jnp.broadcast_to(pidx, (R, padn))], axis=1)
    ev_p = jnp.concatenate(
        [edge_val, jnp.zeros((R, padn), jnp.float32)], axis=1)

    h = _sc_aggregate(xu_all.reshape(V * R, HID), xv_all.reshape(U * R, HID),
                      er_p.reshape(-1), ec_p.reshape(-1), ev_p.reshape(-1))

    r1 = lambda x: x.reshape(1, -1)
    h5 = h.reshape(2, 2, R, U, HID)
    p0u, p0v, su, sv = _combine_a(
        h5, xv_all, xu_all, side_feature_u, side_feature_v,
        W_side_u, W_side_v, W_cat_u, W_cat_v)
    eu, ev = _combine_b(
        p0u, p0v, su, sv,
        r1(g_side_u), r1(be_side_u), r1(g_side_v), r1(be_side_v),
        W_cat_u[2 * RH:], W_cat_v[2 * RH:],
        r1(g_cat_u), r1(be_cat_u), r1(g_cat_v), r1(be_cat_v))
    return _score(eu, Q, ev)
